# baseline (speedup 1.0000x reference)
import functools

import jax
import jax.numpy as jnp
from jax import lax
from jax.experimental import pallas as pl
from jax.experimental.pallas import tpu as pltpu
from jax.experimental.pallas import tpu_sc as plsc

_NC = 2
_NS = 16
_NW = _NC * _NS
_D = 64
_SUB = 256


@functools.cache
def _make_lookup(n_tok: int, n_pos: int):
    b_per_w = n_tok // _NW
    n_sub = b_per_w * n_pos // _SUB
    n_grp = b_per_w * n_pos // 128
    assert n_sub % 2 == 0 and b_per_w == 2 * _SUB
    mesh = plsc.VectorSubcoreMesh(core_axis_name="c", subcore_axis_name="s")

    @functools.partial(
        pl.kernel,
        mesh=mesh,
        compiler_params=pltpu.CompilerParams(use_tc_tiling_on_sc=False,
                                             needs_layout_passes=False),
        out_type=jax.ShapeDtypeStruct((n_pos, 8, n_tok // 128, 8, 128),
                                      jnp.float32),
        scratch_types=[
            pltpu.VMEM((n_grp, 128), jnp.int32),
            pltpu.VMEM((2, _SUB, _D), jnp.float32),
            pltpu.VMEM((2, 8, 2, 8, 128), jnp.float32),
            pltpu.SemaphoreType.DMA,
            pltpu.SemaphoreType.DMA,
            pltpu.SemaphoreType.DMA,
            pltpu.SemaphoreType.DMA,
        ],
    )
    def lookup(table_hbm, idx_hbm, out_hbm,
               idx_v, rows_v, trans_v, sg0, sg1, so0, so1):
        wid = lax.axis_index("s") * _NC + lax.axis_index("c")
        pltpu.sync_copy(idx_hbm.at[wid], idx_v)
        sg = (sg0, sg1)
        so = (so0, so1)

        def fire_g(s, b):
            for g in range(2):
                pltpu.async_copy(
                    table_hbm.at[idx_v.at[2 * s + g]],
                    rows_v.at[b].at[pl.ds(g * 128, 128)],
                    sg[b],
                )

        def wait_g(b):
            pltpu.make_async_copy(
                table_hbm.at[pl.ds(0, _SUB)], rows_v.at[b], sg[b]
            ).wait()

        def transpose(b):
            @plsc.parallel_loop(0, _SUB, unroll=8)
            def j_body(j):
                tc16 = jnp.full((16,), 0, jnp.int32) + (j >> 7)
                jj16 = jnp.full((16,), 0, jnp.int32) + lax.bitwise_and(j, 127)
                for dd in range(4):
                    dvec = dd * 16 + lax.iota(jnp.int32, 16)
                    v16 = rows_v[b, j, pl.ds(dd * 16, 16)]
                    plsc.store_scatter(
                        trans_v.at[b],
                        [lax.shift_right_logical(dvec, 3), tc16,
                         lax.bitwise_and(dvec, 7), jj16],
                        v16)

        def fire_o(t, tc0, b):
            pltpu.async_copy(
                trans_v.at[b], out_hbm.at[t, :, pl.ds(tc0, 2)], so[b])

        def wait_o(b):
            pltpu.make_async_copy(
                trans_v.at[b], out_hbm.at[0, :, pl.ds(0, 2)], so[b]
            ).wait()

        def body(i, carry):
            fire_g(2 * i, 0)

            @pl.when(i > 1)
            def _():
                wait_o(1)

            @pl.when(i > 0)
            def _():
                wait_g(1)
                transpose(1)
                fire_o(i - 1, wid * 4 + 2, 1)

            fire_g(2 * i + 1, 1)
            wait_g(0)

            @pl.when(i > 0)
            def _():
                wait_o(0)

            transpose(0)
            fire_o(i, wid * 4, 0)
            return carry

        lax.fori_loop(0, n_sub // 2, body, 0)
        wait_g(1)
        wait_o(1)
        transpose(1)
        fire_o(n_sub // 2 - 1, wid * 4 + 2, 1)
        wait_o(0)
        wait_o(1)

    return lookup


def kernel(token_ids, weight):
    n_tok, n_pos = token_ids.shape
    b_per_w = n_tok // _NW
    idx = (token_ids.astype(jnp.int32)
           .reshape(_NW, b_per_w, n_pos)
           .transpose(0, 2, 1)
           .reshape(_NW, b_per_w * n_pos // 128, 128))
    out5 = _make_lookup(n_tok, n_pos)(weight, idx)
    return (out5.transpose(2, 4, 0, 1, 3)
            .reshape(n_tok, n_pos, _D))


# speedup vs baseline: 1.0296x; 1.0296x over previous
import functools

import jax
import jax.numpy as jnp
from jax import lax
from jax.experimental import pallas as pl
from jax.experimental.pallas import tpu as pltpu
from jax.experimental.pallas import tpu_sc as plsc

_NC = 2
_NS = 16
_NW = _NC * _NS
_D = 64
_SUB = 256


@functools.cache
def _make_lookup(n_tok: int, n_pos: int):
    b_per_w = n_tok // _NW
    n_sub = b_per_w * n_pos // _SUB
    n_grp = b_per_w * n_pos // 128
    assert n_sub % 2 == 0 and b_per_w == 2 * _SUB
    mesh = plsc.VectorSubcoreMesh(core_axis_name="c", subcore_axis_name="s")

    @functools.partial(
        pl.kernel,
        mesh=mesh,
        compiler_params=pltpu.CompilerParams(use_tc_tiling_on_sc=False,
                                             needs_layout_passes=False),
        out_type=jax.ShapeDtypeStruct((n_pos, 8, n_tok // 128, 8, 128),
                                      jnp.float32),
        scratch_types=[
            pltpu.VMEM((n_grp, 128), jnp.int32),
            pltpu.VMEM((2, _SUB, _D), jnp.float32),
            pltpu.VMEM((2, 8, 2, 8, 128), jnp.float32),
            pltpu.SemaphoreType.DMA,
            pltpu.SemaphoreType.DMA,
            pltpu.SemaphoreType.DMA,
            pltpu.SemaphoreType.DMA,
        ],
    )
    def lookup(table_hbm, idx_hbm, out_hbm,
               idx_v, rows_v, trans_v, sg0, sg1, so0, so1):
        wid = lax.axis_index("s") * _NC + lax.axis_index("c")
        pltpu.sync_copy(idx_hbm.at[wid], idx_v)
        sg = (sg0, sg1)
        so = (so0, so1)

        def fire_g(s, b):
            for g in range(2):
                pltpu.async_copy(
                    table_hbm.at[idx_v.at[2 * s + g]],
                    rows_v.at[b].at[pl.ds(g * 128, 128)],
                    sg[b],
                )

        def wait_g(b):
            pltpu.make_async_copy(
                table_hbm.at[pl.ds(0, _SUB)], rows_v.at[b], sg[b]
            ).wait()

        def transpose(b):
            @plsc.parallel_loop(0, 16, unroll=8)
            def jg_body(jg):
                row16 = lax.iota(jnp.int32, 16) + jg * 16
                tc = jg // 8
                j0 = lax.rem(jg, 8) * 16
                for d8 in range(0, _D, 8):
                    xs = [
                        plsc.load_gather(
                            rows_v.at[b],
                            [row16, jnp.full((16,), d8 + k, jnp.int32)])
                        for k in range(8)
                    ]
                    for k in range(8):
                        d = d8 + k
                        trans_v[b, d // 8, tc, d % 8, pl.ds(j0, 16)] = xs[k]

        def fire_o(t, tc0, b):
            pltpu.async_copy(
                trans_v.at[b], out_hbm.at[t, :, pl.ds(tc0, 2)], so[b])

        def wait_o(b):
            pltpu.make_async_copy(
                trans_v.at[b], out_hbm.at[0, :, pl.ds(0, 2)], so[b]
            ).wait()

        def body(i, carry):
            fire_g(2 * i, 0)

            @pl.when(i > 1)
            def _():
                wait_o(1)

            @pl.when(i > 0)
            def _():
                wait_g(1)
                transpose(1)
                fire_o(i - 1, wid * 4 + 2, 1)

            fire_g(2 * i + 1, 1)
            wait_g(0)

            @pl.when(i > 0)
            def _():
                wait_o(0)

            transpose(0)
            fire_o(i, wid * 4, 0)
            return carry

        lax.fori_loop(0, n_sub // 2, body, 0)
        wait_g(1)
        wait_o(1)
        transpose(1)
        fire_o(n_sub // 2 - 1, wid * 4 + 2, 1)
        wait_o(0)
        wait_o(1)

    return lookup


def kernel(token_ids, weight):
    n_tok, n_pos = token_ids.shape
    b_per_w = n_tok // _NW
    idx = (token_ids.astype(jnp.int32)
           .reshape(_NW, b_per_w, n_pos)
           .transpose(0, 2, 1)
           .reshape(_NW, b_per_w * n_pos // 128, 128))
    out5 = _make_lookup(n_tok, n_pos)(weight, idx)
    return (out5.transpose(2, 4, 0, 1, 3)
            .reshape(n_tok, n_pos, _D))


# speedup vs baseline: 1.6631x; 1.6153x over previous
import functools

import jax
import jax.numpy as jnp
from jax import lax
from jax.experimental import pallas as pl
from jax.experimental.pallas import tpu as pltpu
from jax.experimental.pallas import tpu_sc as plsc

_NC = 2
_NS = 16
_NW = _NC * _NS
_D = 64
_SUB = 256


@functools.cache
def _make_lookup(n_tok: int, n_pos: int):
    b_per_w = n_tok // _NW
    n_sub = b_per_w * n_pos // _SUB
    n_grp = b_per_w * n_pos // 128
    assert n_sub % 2 == 0 and b_per_w == 2 * _SUB
    mesh = plsc.VectorSubcoreMesh(core_axis_name="c", subcore_axis_name="s")

    @functools.partial(
        pl.kernel,
        mesh=mesh,
        compiler_params=pltpu.CompilerParams(use_tc_tiling_on_sc=False,
                                             needs_layout_passes=False),
        out_type=jax.ShapeDtypeStruct((n_pos, 8, n_tok // 128, 8, 128),
                                      jnp.float32),
        scratch_types=[
            pltpu.VMEM((n_grp, 128), jnp.int32),
            pltpu.VMEM((2, _SUB, _D), jnp.float32),
            pltpu.VMEM((2, 8, 2, 8, 128), jnp.float32),
            pltpu.SemaphoreType.DMA,
            pltpu.SemaphoreType.DMA,
            pltpu.SemaphoreType.DMA,
            pltpu.SemaphoreType.DMA,
        ],
    )
    def lookup(table_hbm, idx_hbm, out_hbm,
               idx_v, rows_v, trans_v, sg0, sg1, so0, so1):
        wid = lax.axis_index("s") * _NC + lax.axis_index("c")
        pltpu.sync_copy(idx_hbm.at[wid], idx_v)
        sg = (sg0, sg1)
        so = (so0, so1)

        def fire_g(s, b):
            for g in range(2):
                pltpu.async_copy(
                    table_hbm.at[idx_v.at[2 * s + g]],
                    rows_v.at[b].at[pl.ds(g * 128, 128)],
                    sg[b],
                )

        def wait_g(b):
            pltpu.make_async_copy(
                table_hbm.at[pl.ds(0, _SUB)], rows_v.at[b], sg[b]
            ).wait()

        def transpose(b):
            @plsc.parallel_loop(0, 16, unroll=4)
            def jg_body(jg):
                iota = lax.iota(jnp.int32, 16)
                row16 = iota + jg * 16
                tc16 = jnp.full((16,), 0, jnp.int32) + jg // 8
                jj16 = iota + lax.rem(jg, 8) * 16
                for d in range(_D):
                    dcol16 = lax.bitwise_and(iota + d, 63)
                    x = plsc.load_gather(rows_v.at[b], [row16, dcol16])
                    plsc.store_scatter(
                        trans_v.at[b],
                        [lax.shift_right_logical(dcol16, 3), tc16,
                         lax.bitwise_and(dcol16, 7), jj16],
                        x)

        def fire_o(t, tc0, b):
            pltpu.async_copy(
                trans_v.at[b], out_hbm.at[t, :, pl.ds(tc0, 2)], so[b])

        def wait_o(b):
            pltpu.make_async_copy(
                trans_v.at[b], out_hbm.at[0, :, pl.ds(0, 2)], so[b]
            ).wait()

        def body(i, carry):
            fire_g(2 * i, 0)

            @pl.when(i > 1)
            def _():
                wait_o(1)

            @pl.when(i > 0)
            def _():
                wait_g(1)
                transpose(1)
                fire_o(i - 1, wid * 4 + 2, 1)

            fire_g(2 * i + 1, 1)
            wait_g(0)

            @pl.when(i > 0)
            def _():
                wait_o(0)

            transpose(0)
            fire_o(i, wid * 4, 0)
            return carry

        lax.fori_loop(0, n_sub // 2, body, 0)
        wait_g(1)
        wait_o(1)
        transpose(1)
        fire_o(n_sub // 2 - 1, wid * 4 + 2, 1)
        wait_o(0)
        wait_o(1)

    return lookup


def kernel(token_ids, weight):
    n_tok, n_pos = token_ids.shape
    b_per_w = n_tok // _NW
    idx = (token_ids.astype(jnp.int32)
           .reshape(_NW, b_per_w, n_pos)
           .transpose(0, 2, 1)
           .reshape(_NW, b_per_w * n_pos // 128, 128))
    out5 = _make_lookup(n_tok, n_pos)(weight, idx)
    return (out5.transpose(2, 4, 0, 1, 3)
            .reshape(n_tok, n_pos, _D))
